# baseline (speedup 1.0000x reference)
import functools

import jax
import jax.numpy as jnp
from jax import lax
from jax.experimental import pallas as pl
from jax.experimental.pallas import tpu as pltpu
from jax.experimental.pallas import tpu_sc as plsc

N = 10000
NP = 10240
E = 320000
CH = 128
K = 2
AGG_F = 160
AGG_SP = 4
DEG_F = 128
DEG_S = 32
NCH_TOT = 16 * AGG_F
CH_ALLOC = NCH_TOT + 16 * AGG_SP
EP = CH_ALLOC * CH
IN_F = 128
H = 64
G = 64
C = 2
RPT = NP // 16
BLK = 1024
F32 = jnp.float32

_mesh = plsc.VectorSubcoreMesh(
    core_axis_name="c", subcore_axis_name="s", num_cores=2, num_subcores=16)


def _deg_body(dstr_hbm, out_hbm, dst2d, ones_v, zero_v, acc, ssem):
    c = lax.axis_index("c")
    s = lax.axis_index("s")
    for i in range(CH // 16):
        ones_v[pl.ds(i * 16, 16)] = jnp.ones((16,), F32)
    for i in range(RPT // 16):
        zero_v[pl.ds(i * 16, 16)] = jnp.zeros((16,), F32)
    pltpu.sync_copy(zero_v, acc.at[pl.ds(s * RPT, RPT)])
    base = jnp.where(c == 0, s * DEG_F, 16 * DEG_F + s * DEG_S)
    nbatch = jnp.where(c == 0, DEG_F // 8, DEG_S // 8)

    @pl.when(c == 0)
    def _():
        pltpu.sync_copy(dstr_hbm.at[pl.ds(base, DEG_F)], dst2d)

    @pl.when(c == 1)
    def _():
        pltpu.sync_copy(dstr_hbm.at[pl.ds(base, DEG_S)],
                        dst2d.at[pl.ds(0, DEG_S)])

    plsc.subcore_barrier()

    def body(g, carry):
        for b in range(8):
            pltpu.async_copy(ones_v, acc.at[dst2d.at[g * 8 + b]], ssem, add=True)
        for b in range(8):
            pltpu.make_async_copy(ones_v, acc.at[dst2d.at[0]], ssem).wait()
        return carry

    lax.fori_loop(0, nbatch, body, 0)
    plsc.subcore_barrier()
    pltpu.sync_copy(acc.at[pl.ds(s * RPT, RPT)], out_hbm.at[c, pl.ds(s * RPT, RPT)])


_deg_call = pl.kernel(
    _deg_body,
    out_type=jax.ShapeDtypeStruct((2, NP), F32),
    mesh=_mesh,
    scratch_types=[
        pltpu.VMEM((DEG_F, CH), jnp.int32),
        pltpu.VMEM((CH,), F32),
        pltpu.VMEM((RPT,), F32),
        pltpu.VMEM_SHARED((NP,), F32),
        pltpu.SemaphoreType.DMA,
    ],
    compiler_params=pltpu.CompilerParams(use_tc_tiling_on_sc=False),
    name="sc_degree",
)


def _agg_body(y_hbm, srcr_hbm, dstr_hbm, out_hbm, src2d, dst2d, rows_v, zrow_v,
              acc, gsem0, gsem1, ssem0, ssem1):
    c = lax.axis_index("c")
    s = lax.axis_index("s")
    gsem = (gsem0, gsem1)
    ssem = (ssem0, ssem1)

    for i in range(16):
        for f in range(H // 16):
            zrow_v[i, pl.ds(f * 16, 16)] = jnp.zeros((16,), F32)

    def zbody(j, carry):
        pltpu.sync_copy(zrow_v, acc.at[pl.ds(s * RPT + j * 16, 16)])
        return carry

    lax.fori_loop(0, RPT // 16, zbody, 0)

    @pl.when(c == 0)
    def _():
        pltpu.sync_copy(srcr_hbm.at[pl.ds(s * AGG_F, AGG_F)], src2d)
        pltpu.sync_copy(dstr_hbm.at[pl.ds(s * AGG_F, AGG_F)], dst2d)

    @pl.when(c == 1)
    def _():
        base = 16 * AGG_F + s * AGG_SP
        pltpu.sync_copy(srcr_hbm.at[pl.ds(base, AGG_SP)],
                        src2d.at[pl.ds(0, AGG_SP)])
        pltpu.sync_copy(dstr_hbm.at[pl.ds(base, AGG_SP)],
                        dst2d.at[pl.ds(0, AGG_SP)])

    plsc.subcore_barrier()

    def fire(g, m):
        for b in range(K):
            pltpu.async_copy(y_hbm.at[src2d.at[g * K + b]], rows_v.at[m, b],
                             gsem[m])

    def group(g, m, do_fire):
        for b in range(K):
            pltpu.make_async_copy(y_hbm.at[src2d.at[0]], rows_v.at[m, b],
                                  gsem[m]).wait()
        for b in range(K):
            pltpu.async_copy(rows_v.at[m, b], acc.at[dst2d.at[g * K + b]],
                             ssem[m], add=True)
        for b in range(K):
            pltpu.make_async_copy(rows_v.at[m, b], acc.at[dst2d.at[0]],
                                  ssem[m]).wait()
        if do_fire:
            fire(g + 2, m)

    fire(0, 0)
    fire(1, 1)
    ng = jnp.where(c == 0, AGG_F // K, AGG_SP // K)

    def body(i, carry):
        group(2 * i, 0, True)
        group(2 * i + 1, 1, True)
        return carry

    lax.fori_loop(0, ng // 2 - 1, body, 0)
    group(ng - 2, 0, False)
    group(ng - 1, 1, False)

    plsc.subcore_barrier()

    @pl.when(c == 0)
    def _():
        pltpu.sync_copy(acc.at[pl.ds(s * RPT, RPT)],
                        out_hbm.at[pl.ds(s * RPT, RPT)])


_agg_call = pl.kernel(
    _agg_body,
    out_type=jax.ShapeDtypeStruct((NP, H), F32),
    mesh=_mesh,
    scratch_types=[
        pltpu.VMEM((AGG_F, CH), jnp.int32),
        pltpu.VMEM((AGG_F, CH), jnp.int32),
        pltpu.VMEM((2, K, CH, H), F32),
        pltpu.VMEM((16, H), F32),
        pltpu.VMEM_SHARED((NP, H), F32),
        pltpu.SemaphoreType.DMA,
        pltpu.SemaphoreType.DMA,
        pltpu.SemaphoreType.DMA,
        pltpu.SemaphoreType.DMA,
    ],
    compiler_params=pltpu.CompilerParams(use_tc_tiling_on_sc=False),
    name="sc_aggregate",
)


def _mm1_body(x_ref, w_ref, degT_ref, y_ref, dis_ref):
    d = degT_ref[:, 0:1] + degT_ref[:, 1:2] + 1.0
    dis = lax.rsqrt(d)
    xw = jnp.dot(x_ref[...], w_ref[...], preferred_element_type=F32)
    y_ref[...] = dis * xw
    dis_ref[...] = dis


_mm1 = pl.pallas_call(
    _mm1_body,
    grid=(NP // BLK,),
    in_specs=[
        pl.BlockSpec((BLK, IN_F), lambda i: (i, 0)),
        pl.BlockSpec((IN_F, H), lambda i: (0, 0)),
        pl.BlockSpec((BLK, 2), lambda i: (i, 0)),
    ],
    out_specs=[
        pl.BlockSpec((BLK, H), lambda i: (i, 0)),
        pl.BlockSpec((BLK, 1), lambda i: (i, 0)),
    ],
    out_shape=[
        jax.ShapeDtypeStruct((NP, H), F32),
        jax.ShapeDtypeStruct((NP, 1), F32),
    ],
)


def _mm2_body(agg_ref, y1_ref, dis_ref, b1_ref, w2_ref, y2_ref):
    dis = dis_ref[...]
    h1 = jnp.maximum(
        dis * (agg_ref[...] + y1_ref[...]) + b1_ref[...], 0.0)
    y2_ref[...] = dis * jnp.dot(h1, w2_ref[...], preferred_element_type=F32)


_mm2 = pl.pallas_call(
    _mm2_body,
    grid=(NP // BLK,),
    in_specs=[
        pl.BlockSpec((BLK, H), lambda i: (i, 0)),
        pl.BlockSpec((BLK, H), lambda i: (i, 0)),
        pl.BlockSpec((BLK, 1), lambda i: (i, 0)),
        pl.BlockSpec((1, H), lambda i: (0, 0)),
        pl.BlockSpec((H, H), lambda i: (0, 0)),
    ],
    out_specs=pl.BlockSpec((BLK, H), lambda i: (i, 0)),
    out_shape=jax.ShapeDtypeStruct((NP, H), F32),
)


def _pool_body(agg_ref, y2_ref, dis_ref, b2_ref, batch_ref, pooled_ref, cnt_ref):
    i = pl.program_id(0)
    dis = dis_ref[...]
    h2 = jnp.maximum(
        dis * (agg_ref[...] + y2_ref[...]) + b2_ref[...], 0.0)
    gids = lax.broadcasted_iota(jnp.int32, (1, G), 1)
    sel = (batch_ref[...] == gids).astype(F32)
    psum = lax.dot_general(sel, h2, (((0,), (0,)), ((), ())),
                           preferred_element_type=F32)
    csum = lax.dot_general(sel, jnp.ones((BLK, 1), F32),
                           (((0,), (0,)), ((), ())),
                           preferred_element_type=F32)

    @pl.when(i == 0)
    def _():
        pooled_ref[...] = jnp.zeros_like(pooled_ref)
        cnt_ref[...] = jnp.zeros_like(cnt_ref)

    pooled_ref[...] += psum
    cnt_ref[...] += csum


_pool = pl.pallas_call(
    _pool_body,
    grid=(NP // BLK,),
    in_specs=[
        pl.BlockSpec((BLK, H), lambda i: (i, 0)),
        pl.BlockSpec((BLK, H), lambda i: (i, 0)),
        pl.BlockSpec((BLK, 1), lambda i: (i, 0)),
        pl.BlockSpec((1, H), lambda i: (0, 0)),
        pl.BlockSpec((BLK, 1), lambda i: (i, 0)),
    ],
    out_specs=[
        pl.BlockSpec((G, H), lambda i: (0, 0)),
        pl.BlockSpec((G, 1), lambda i: (0, 0)),
    ],
    out_shape=[
        jax.ShapeDtypeStruct((G, H), F32),
        jax.ShapeDtypeStruct((G, 1), F32),
    ],
)


def _head_body(pooled_ref, cnt_ref, wc1_ref, bc1_ref, wc2_ref, bc2_ref, out_ref):
    mean = pooled_ref[...] / jnp.maximum(cnt_ref[...], 1.0)
    hc = jnp.maximum(
        jnp.dot(mean, wc1_ref[...], preferred_element_type=F32) + bc1_ref[...],
        0.0)
    out_ref[...] = (jnp.dot(hc, wc2_ref[...], preferred_element_type=F32)
                    + bc2_ref[...])


_head = pl.pallas_call(
    _head_body,
    out_shape=jax.ShapeDtypeStruct((G, C), F32),
)


def kernel(x, edge_index, batch, W1, b1, W2, b2, Wc1, bc1, Wc2, bc2):
    src = edge_index[0]
    dst = edge_index[1]
    pad_e = EP - E
    src_p = jnp.concatenate([src, jnp.zeros((pad_e,), jnp.int32)])
    src_p = src_p.reshape(CH_ALLOC, CH)
    dst_p = jnp.concatenate([dst, jnp.full((pad_e,), N, jnp.int32)])
    dst_p = dst_p.reshape(CH_ALLOC, CH)
    x_p = jnp.pad(x, ((0, NP - N), (0, 0)))
    batch_p = jnp.concatenate(
        [batch, jnp.full((NP - N,), G, jnp.int32)]).reshape(NP, 1)

    deg = _deg_call(dst_p)
    y1, dis = _mm1(x_p, W1, deg.T)
    agg1 = _agg_call(y1, src_p, dst_p)
    y2 = _mm2(agg1, y1, dis, b1.reshape(1, H), W2)
    agg2 = _agg_call(y2, src_p, dst_p)
    pooled, cnt = _pool(agg2, y2, dis, b2.reshape(1, H), batch_p)
    logits = _head(pooled, cnt, Wc1, bc1.reshape(1, H // 2),
                   Wc2, bc2.reshape(1, C))
    return logits


# speedup vs baseline: 1.8110x; 1.8110x over previous
import functools

import jax
import jax.numpy as jnp
from jax import lax
from jax.experimental import pallas as pl
from jax.experimental.pallas import tpu as pltpu
from jax.experimental.pallas import tpu_sc as plsc

N = 10000
NP = 10240
E = 320000
CH = 128
K = 2
AGG_F = 148
AGG_S = 12
DEG_F = 128
DEG_S = 32
NCH_TOT = 16 * (AGG_F + AGG_S)
CH_ALLOC = NCH_TOT
EP = CH_ALLOC * CH
IN_F = 128
H = 64
G = 64
C = 2
RPT = NP // 16
BLK = 1024
F32 = jnp.float32

_mesh = plsc.VectorSubcoreMesh(
    core_axis_name="c", subcore_axis_name="s", num_cores=2, num_subcores=16)


def _deg_body(dstr_hbm, out_hbm, dst2d, ones_v, zero_v, acc, ssem):
    c = lax.axis_index("c")
    s = lax.axis_index("s")
    for i in range(CH // 16):
        ones_v[pl.ds(i * 16, 16)] = jnp.ones((16,), F32)
    for i in range(RPT // 16):
        zero_v[pl.ds(i * 16, 16)] = jnp.zeros((16,), F32)
    pltpu.sync_copy(zero_v, acc.at[pl.ds(s * RPT, RPT)])
    base = jnp.where(c == 0, s * DEG_F, 16 * DEG_F + s * DEG_S)
    nbatch = jnp.where(c == 0, DEG_F // 8, DEG_S // 8)

    @pl.when(c == 0)
    def _():
        pltpu.sync_copy(dstr_hbm.at[pl.ds(base, DEG_F)], dst2d)

    @pl.when(c == 1)
    def _():
        pltpu.sync_copy(dstr_hbm.at[pl.ds(base, DEG_S)],
                        dst2d.at[pl.ds(0, DEG_S)])

    plsc.subcore_barrier()

    def body(g, carry):
        for b in range(8):
            pltpu.async_copy(ones_v, acc.at[dst2d.at[g * 8 + b]], ssem, add=True)
        for b in range(8):
            pltpu.make_async_copy(ones_v, acc.at[dst2d.at[0]], ssem).wait()
        return carry

    lax.fori_loop(0, nbatch, body, 0)
    plsc.subcore_barrier()
    pltpu.sync_copy(acc.at[pl.ds(s * RPT, RPT)], out_hbm.at[c, pl.ds(s * RPT, RPT)])


_deg_call = pl.kernel(
    _deg_body,
    out_type=jax.ShapeDtypeStruct((2, NP), F32),
    mesh=_mesh,
    scratch_types=[
        pltpu.VMEM((DEG_F, CH), jnp.int32),
        pltpu.VMEM((CH,), F32),
        pltpu.VMEM((RPT,), F32),
        pltpu.VMEM_SHARED((NP,), F32),
        pltpu.SemaphoreType.DMA,
    ],
    compiler_params=pltpu.CompilerParams(use_tc_tiling_on_sc=False),
    name="sc_degree",
)


def _agg_body(y_hbm, srcr_hbm, dstr_hbm, out_hbm, src2d, dst2d, rows_v, zrow_v,
              acc, gsem0, gsem1, ssem0, ssem1, zsem):
    c = lax.axis_index("c")
    s = lax.axis_index("s")
    gsem = (gsem0, gsem1)
    ssem = (ssem0, ssem1)

    for i in range(16):
        for f in range(H // 16):
            zrow_v[i, pl.ds(f * 16, 16)] = jnp.zeros((16,), F32)

    def zbody(j, carry):
        for b in range(8):
            pltpu.async_copy(
                zrow_v, acc.at[pl.ds(s * RPT + (j * 8 + b) * 16, 16)], zsem)
        for b in range(8):
            pltpu.make_async_copy(zrow_v, acc.at[pl.ds(0, 16)], zsem).wait()
        return carry

    lax.fori_loop(0, RPT // 128, zbody, 0)

    @pl.when(c == 0)
    def _():
        pltpu.sync_copy(srcr_hbm.at[pl.ds(s * AGG_F, AGG_F)], src2d)
        pltpu.sync_copy(dstr_hbm.at[pl.ds(s * AGG_F, AGG_F)], dst2d)

    @pl.when(c == 1)
    def _():
        base = 16 * AGG_F + s * AGG_S
        pltpu.sync_copy(srcr_hbm.at[pl.ds(base, AGG_S)],
                        src2d.at[pl.ds(0, AGG_S)])
        pltpu.sync_copy(dstr_hbm.at[pl.ds(base, AGG_S)],
                        dst2d.at[pl.ds(0, AGG_S)])

    plsc.subcore_barrier()

    def fire(g, m):
        for b in range(K):
            pltpu.async_copy(y_hbm.at[src2d.at[g * K + b]], rows_v.at[m, b],
                             gsem[m])

    def group(g, m, do_fire):
        for b in range(K):
            pltpu.make_async_copy(y_hbm.at[src2d.at[0]], rows_v.at[m, b],
                                  gsem[m]).wait()
        for b in range(K):
            pltpu.async_copy(rows_v.at[m, b], acc.at[dst2d.at[g * K + b]],
                             ssem[m], add=True)
        for b in range(K):
            pltpu.make_async_copy(rows_v.at[m, b], acc.at[dst2d.at[0]],
                                  ssem[m]).wait()
        if do_fire:
            fire(g + 2, m)

    fire(0, 0)
    fire(1, 1)
    ng = jnp.where(c == 0, AGG_F // K, AGG_S // K)

    def body(i, carry):
        group(2 * i, 0, True)
        group(2 * i + 1, 1, True)
        return carry

    lax.fori_loop(0, ng // 2 - 1, body, 0)
    group(ng - 2, 0, False)
    group(ng - 1, 1, False)

    plsc.subcore_barrier()
    pltpu.sync_copy(acc.at[pl.ds(s * RPT, RPT)],
                    out_hbm.at[c, pl.ds(s * RPT, RPT)])


_agg_call = pl.kernel(
    _agg_body,
    out_type=jax.ShapeDtypeStruct((2, NP, H), F32),
    mesh=_mesh,
    scratch_types=[
        pltpu.VMEM((AGG_F, CH), jnp.int32),
        pltpu.VMEM((AGG_F, CH), jnp.int32),
        pltpu.VMEM((2, K, CH, H), F32),
        pltpu.VMEM((16, H), F32),
        pltpu.VMEM_SHARED((NP, H), F32),
        pltpu.SemaphoreType.DMA,
        pltpu.SemaphoreType.DMA,
        pltpu.SemaphoreType.DMA,
        pltpu.SemaphoreType.DMA,
        pltpu.SemaphoreType.DMA,
    ],
    compiler_params=pltpu.CompilerParams(use_tc_tiling_on_sc=False),
    name="sc_aggregate",
)


def _mm1_body(x_ref, w_ref, degT_ref, y_ref, dis_ref):
    d = degT_ref[:, 0:1] + degT_ref[:, 1:2] + 1.0
    dis = lax.rsqrt(d)
    xw = jnp.dot(x_ref[...], w_ref[...], preferred_element_type=F32)
    y_ref[...] = dis * xw
    dis_ref[...] = dis


_mm1 = pl.pallas_call(
    _mm1_body,
    grid=(NP // BLK,),
    in_specs=[
        pl.BlockSpec((BLK, IN_F), lambda i: (i, 0)),
        pl.BlockSpec((IN_F, H), lambda i: (0, 0)),
        pl.BlockSpec((BLK, 2), lambda i: (i, 0)),
    ],
    out_specs=[
        pl.BlockSpec((BLK, H), lambda i: (i, 0)),
        pl.BlockSpec((BLK, 1), lambda i: (i, 0)),
    ],
    out_shape=[
        jax.ShapeDtypeStruct((NP, H), F32),
        jax.ShapeDtypeStruct((NP, 1), F32),
    ],
)


def _mm2_body(agg_ref, y1_ref, dis_ref, b1_ref, w2_ref, y2_ref):
    dis = dis_ref[...]
    h1 = jnp.maximum(
        dis * (agg_ref[0] + agg_ref[1] + y1_ref[...]) + b1_ref[...], 0.0)
    y2_ref[...] = dis * jnp.dot(h1, w2_ref[...], preferred_element_type=F32)


_mm2 = pl.pallas_call(
    _mm2_body,
    grid=(NP // BLK,),
    in_specs=[
        pl.BlockSpec((2, BLK, H), lambda i: (0, i, 0)),
        pl.BlockSpec((BLK, H), lambda i: (i, 0)),
        pl.BlockSpec((BLK, 1), lambda i: (i, 0)),
        pl.BlockSpec((1, H), lambda i: (0, 0)),
        pl.BlockSpec((H, H), lambda i: (0, 0)),
    ],
    out_specs=pl.BlockSpec((BLK, H), lambda i: (i, 0)),
    out_shape=jax.ShapeDtypeStruct((NP, H), F32),
)


def _pool_body(agg_ref, y2_ref, dis_ref, b2_ref, batch_ref, pooled_ref, cnt_ref):
    i = pl.program_id(0)
    dis = dis_ref[...]
    h2 = jnp.maximum(
        dis * (agg_ref[0] + agg_ref[1] + y2_ref[...]) + b2_ref[...], 0.0)
    gids = lax.broadcasted_iota(jnp.int32, (1, G), 1)
    sel = (batch_ref[...] == gids).astype(F32)
    psum = lax.dot_general(sel, h2, (((0,), (0,)), ((), ())),
                           preferred_element_type=F32)
    csum = lax.dot_general(sel, jnp.ones((BLK, 1), F32),
                           (((0,), (0,)), ((), ())),
                           preferred_element_type=F32)

    @pl.when(i == 0)
    def _():
        pooled_ref[...] = jnp.zeros_like(pooled_ref)
        cnt_ref[...] = jnp.zeros_like(cnt_ref)

    pooled_ref[...] += psum
    cnt_ref[...] += csum


_pool = pl.pallas_call(
    _pool_body,
    grid=(NP // BLK,),
    in_specs=[
        pl.BlockSpec((2, BLK, H), lambda i: (0, i, 0)),
        pl.BlockSpec((BLK, H), lambda i: (i, 0)),
        pl.BlockSpec((BLK, 1), lambda i: (i, 0)),
        pl.BlockSpec((1, H), lambda i: (0, 0)),
        pl.BlockSpec((BLK, 1), lambda i: (i, 0)),
    ],
    out_specs=[
        pl.BlockSpec((G, H), lambda i: (0, 0)),
        pl.BlockSpec((G, 1), lambda i: (0, 0)),
    ],
    out_shape=[
        jax.ShapeDtypeStruct((G, H), F32),
        jax.ShapeDtypeStruct((G, 1), F32),
    ],
)


def _head_body(pooled_ref, cnt_ref, wc1_ref, bc1_ref, wc2_ref, bc2_ref, out_ref):
    mean = pooled_ref[...] / jnp.maximum(cnt_ref[...], 1.0)
    hc = jnp.maximum(
        jnp.dot(mean, wc1_ref[...], preferred_element_type=F32) + bc1_ref[...],
        0.0)
    out_ref[...] = (jnp.dot(hc, wc2_ref[...], preferred_element_type=F32)
                    + bc2_ref[...])


_head = pl.pallas_call(
    _head_body,
    out_shape=jax.ShapeDtypeStruct((G, C), F32),
)


def kernel(x, edge_index, batch, W1, b1, W2, b2, Wc1, bc1, Wc2, bc2):
    src = edge_index[0]
    dst = edge_index[1]
    pad_e = EP - E
    src_p = jnp.concatenate([src, jnp.zeros((pad_e,), jnp.int32)])
    src_p = src_p.reshape(CH_ALLOC, CH)
    dst_p = jnp.concatenate([dst, jnp.full((pad_e,), N, jnp.int32)])
    dst_p = dst_p.reshape(CH_ALLOC, CH)
    x_p = jnp.pad(x, ((0, NP - N), (0, 0)))
    batch_p = jnp.concatenate(
        [batch, jnp.full((NP - N,), G, jnp.int32)]).reshape(NP, 1)

    deg = _deg_call(dst_p)
    y1, dis = _mm1(x_p, W1, deg.T)
    agg1 = _agg_call(y1, src_p, dst_p)
    y2 = _mm2(agg1, y1, dis, b1.reshape(1, H), W2)
    agg2 = _agg_call(y2, src_p, dst_p)
    pooled, cnt = _pool(agg2, y2, dis, b2.reshape(1, H), batch_p)
    logits = _head(pooled, cnt, Wc1, bc1.reshape(1, H // 2),
                   Wc2, bc2.reshape(1, C))
    return logits
